# baseline (speedup 1.0000x reference)
import functools

import jax
import jax.numpy as jnp
from jax import lax
from jax.experimental import pallas as pl
from jax.experimental.pallas import tpu as pltpu
from jax.experimental.pallas import tpu_sc as plsc

N = 10000
D = 128
H = 128
C = 2
E = 320000

NC = 2
NS = 16
NW = NC * NS

NPAD = 10112
ROWS_PER_TILE = NPAD // NS
EPT = 10240
EPAD = EPT * NW
CHUNK = 128
CHUNKS = EPT // CHUNK
QUADS = CHUNKS // 4
DCHUNK = CHUNK
DCHUNKS = CHUNKS


def _row_slices():
    full = ROWS_PER_TILE // CHUNK
    out = [(k * CHUNK, CHUNK) for k in range(full)]
    rem = ROWS_PER_TILE - full * CHUNK
    if rem:
        out.append((full * CHUNK, rem))
    return out


def _sc_mesh():
    return plsc.VectorSubcoreMesh(core_axis_name="c", subcore_axis_name="s",
                                  num_cores=NC, num_subcores=NS)


def _deg_body(dst_hbm, ones_hbm, z640_hbm, degp_hbm,
              deg_sh, idx_v, ones_v, z_v, sem, ssem):
    c = lax.axis_index("c")
    s = lax.axis_index("s")
    w = c * NS + s
    base = s * ROWS_PER_TILE
    pltpu.sync_copy(z640_hbm, z_v)
    pltpu.sync_copy(z_v, deg_sh.at[pl.ds(base, ROWS_PER_TILE)])
    pltpu.sync_copy(ones_hbm, ones_v)
    pltpu.sync_copy(dst_hbm.at[w], idx_v)
    plsc.subcore_barrier()

    def group(g, _):
        def fire(b, carry):
            pltpu.async_copy(ones_v, deg_sh.at[idx_v.at[g * 8 + b]], ssem,
                             add=True)
            return carry

        lax.fori_loop(0, 8, fire, 0)

        def drain(b, carry):
            pltpu.make_async_copy(ones_v, deg_sh.at[idx_v.at[g * 8 + b]],
                                  ssem).wait()
            return carry

        lax.fori_loop(0, 8, drain, 0)
        return _

    lax.fori_loop(0, DCHUNKS // 8, group, 0)
    plsc.subcore_barrier()
    pltpu.sync_copy(deg_sh.at[pl.ds(base, ROWS_PER_TILE)], z_v)
    pltpu.sync_copy(z_v, degp_hbm.at[pl.ds(c * NPAD + base, ROWS_PER_TILE)])


def _run_deg(dst3, ones128, z640):
    k = pl.kernel(
        _deg_body,
        out_type=jax.ShapeDtypeStruct((NC * NPAD,), jnp.float32),
        mesh=_sc_mesh(),
        scratch_types=[
            pltpu.VMEM_SHARED((NPAD,), jnp.float32),
            pltpu.VMEM((DCHUNKS, DCHUNK), jnp.int32),
            pltpu.VMEM((DCHUNK,), jnp.float32),
            pltpu.VMEM((ROWS_PER_TILE,), jnp.float32),
            pltpu.SemaphoreType.DMA,
            pltpu.SemaphoreType.DMA,
        ],
    )
    return k(dst3, ones128, z640)


def _msg_body(hs_hbm, dinv_hbm, src2_hbm, dst2_hbm, z2d_hbm,
              accp_hbm, cp_hbm,
              acc_sh, cacc_sh,
              si0, si1, si2, si3, di0, di1, di2, di3,
              rows_a, rows_b, dvals,
              gsa, gsb, ssa, ssb, dsem, isem):
    c = lax.axis_index("c")
    s = lax.axis_index("s")
    w = c * NS + s
    base = s * ROWS_PER_TILE

    def stage(j, sv, dv_):
        pltpu.async_copy(src2_hbm.at[w, pl.ds(j * CHUNK, CHUNK)], sv, isem)
        pltpu.async_copy(dst2_hbm.at[w, pl.ds(j * CHUNK, CHUNK)], dv_, isem)

    def stage_drain(j, sv, dv_):
        pltpu.make_async_copy(src2_hbm.at[w, pl.ds(j * CHUNK, CHUNK)], sv,
                              isem).wait()
        pltpu.make_async_copy(dst2_hbm.at[w, pl.ds(j * CHUNK, CHUNK)], dv_,
                              isem).wait()

    def gather(sv, rbuf, gsem):
        pltpu.async_copy(hs_hbm.at[sv], rbuf, gsem)

    def scat_drain(rbuf, dv_, ssem_):
        pltpu.make_async_copy(rbuf, acc_sh.at[dv_], ssem_).wait()

    def proc(sv, dv_, rbuf, gsem, ssem_):
        pltpu.make_async_copy(hs_hbm.at[sv], rbuf, gsem).wait()
        pltpu.async_copy(rbuf, acc_sh.at[dv_], ssem_, add=True)
        pltpu.async_copy(dinv_hbm.at[dv_], dvals, dsem).wait()
        pltpu.sync_copy(dvals, cacc_sh.at[sv], add=True)

    pltpu.sync_copy(z2d_hbm, rows_a)
    for k in range(8):
        dvals[pl.ds(k * 16, 16)] = jnp.zeros((16,), jnp.float32)
    stage(0, si0, di0)
    stage(1, si1, di1)
    stage(2, si2, di2)
    stage(3, si3, di3)
    for off, sz in _row_slices():
        pltpu.sync_copy(rows_a.at[pl.ds(0, sz)] if sz < CHUNK else rows_a,
                        acc_sh.at[pl.ds(base + off, sz)])
        pltpu.sync_copy(dvals.at[pl.ds(0, sz)] if sz < CHUNK else dvals,
                        cacc_sh.at[pl.ds(base + off, sz)])
    plsc.subcore_barrier()

    stage_drain(0, si0, di0)
    stage_drain(1, si1, di1)
    gather(si0, rows_a, gsa)
    gather(si1, rows_b, gsb)

    def quad(q, _):
        c0 = 4 * q
        proc(si0, di0, rows_a, gsa, ssa)
        proc(si1, di1, rows_b, gsb, ssb)
        stage_drain(c0 + 2, si2, di2)
        stage_drain(c0 + 3, si3, di3)
        scat_drain(rows_a, di0, ssa)
        gather(si2, rows_a, gsa)

        @pl.when(q + 1 < QUADS)
        def _stage01():
            stage(c0 + 4, si0, di0)
            stage(c0 + 5, si1, di1)

        proc(si2, di2, rows_a, gsa, ssa)
        scat_drain(rows_b, di1, ssb)
        gather(si3, rows_b, gsb)
        proc(si3, di3, rows_b, gsb, ssb)
        scat_drain(rows_a, di2, ssa)
        scat_drain(rows_b, di3, ssb)

        @pl.when(q + 1 < QUADS)
        def _refill():
            stage_drain(c0 + 4, si0, di0)
            stage_drain(c0 + 5, si1, di1)
            gather(si0, rows_a, gsa)
            gather(si1, rows_b, gsb)
            stage(c0 + 6, si2, di2)
            stage(c0 + 7, si3, di3)

        return _

    lax.fori_loop(0, QUADS, quad, 0)
    plsc.subcore_barrier()

    for i, (off, sz) in enumerate(_row_slices()):
        r = base + off
        buf = rows_a if i % 2 == 0 else rows_b
        bufs = buf.at[pl.ds(0, sz)] if sz < CHUNK else buf
        pltpu.sync_copy(acc_sh.at[pl.ds(r, sz)], bufs)
        pltpu.sync_copy(bufs, accp_hbm.at[c, pl.ds(r, sz)])
    for off, sz in _row_slices():
        r = base + off
        dvo = dvals.at[pl.ds(0, sz)]
        pltpu.sync_copy(cacc_sh.at[pl.ds(r, sz)], dvo)
        pltpu.sync_copy(dvo, cp_hbm.at[pl.ds(c * NPAD + r, sz)])


def _run_msg(hs, dinv, src2, dst2, z2d):
    k = pl.kernel(
        _msg_body,
        out_type=(
            jax.ShapeDtypeStruct((NC, NPAD, H), jnp.float32),
            jax.ShapeDtypeStruct((NC * NPAD,), jnp.float32),
        ),
        mesh=_sc_mesh(),
        scratch_types=(
            [pltpu.VMEM_SHARED((NPAD, H), jnp.float32),
             pltpu.VMEM_SHARED((NPAD,), jnp.float32)]
            + [pltpu.VMEM((CHUNK,), jnp.int32) for _ in range(8)]
            + [pltpu.VMEM((CHUNK, H), jnp.float32),
               pltpu.VMEM((CHUNK, H), jnp.float32),
               pltpu.VMEM((CHUNK,), jnp.float32)]
            + [pltpu.SemaphoreType.DMA for _ in range(6)]
        ),
    )
    return k(hs, dinv, src2, dst2, z2d)


BLK1 = 632


def _dense1_body(x_ref, w1_ref, degt_ref, hs_ref, dinv_ref):
    pid = pl.program_id(0)
    deg = degt_ref[:, 0:1] + degt_ref[:, 1:2] + 1.0
    row = pid * BLK1 + lax.broadcasted_iota(jnp.int32, (BLK1, 1), 0)
    dinv = jnp.where(row < N, lax.rsqrt(jnp.maximum(deg, 1.0)), 0.0)
    dinv_ref[...] = dinv
    h = jnp.dot(x_ref[...], w1_ref[...], preferred_element_type=jnp.float32)
    hs_ref[...] = h * dinv


def _run_dense1(xp, W1, degT):
    grid = (NPAD // BLK1,)
    return pl.pallas_call(
        _dense1_body,
        grid=grid,
        in_specs=[
            pl.BlockSpec((BLK1, D), lambda i: (i, 0)),
            pl.BlockSpec((D, H), lambda i: (0, 0)),
            pl.BlockSpec((BLK1, NC), lambda i: (i, 0)),
        ],
        out_specs=[
            pl.BlockSpec((BLK1, H), lambda i: (i, 0)),
            pl.BlockSpec((BLK1, 1), lambda i: (i, 0)),
        ],
        out_shape=[
            jax.ShapeDtypeStruct((NPAD, H), jnp.float32),
            jax.ShapeDtypeStruct((NPAD, 1), jnp.float32),
        ],
    )(xp, W1, degT)


BLK2 = 632


def _final_body(accp_ref, hs_ref, dinv_ref, cpt_ref, b1_ref, w2_ref, b2_ref,
                wl_ref, bl_ref, out_ref, g_ref):
    i = pl.program_id(0)

    @pl.when(i == 0)
    def _init():
        g_ref[...] = jnp.zeros_like(g_ref)

    acc = accp_ref[0] + accp_ref[1] + hs_ref[...]
    dinv = dinv_ref[...]
    h1 = jnp.maximum(dinv * acc + b1_ref[...], 0.0)
    cvec = dinv * (cpt_ref[:, 0:1] + cpt_ref[:, 1:2] + dinv)
    g_ref[...] += jnp.sum(h1 * cvec, axis=0, keepdims=True)

    @pl.when(i == pl.num_programs(0) - 1)
    def _fin():
        g = g_ref[...] * (1.0 / N)
        t = jnp.dot(g, w2_ref[...], preferred_element_type=jnp.float32)
        t = t + b2_ref[...]
        out_ref[...] = (
            jnp.dot(t, wl_ref[...], preferred_element_type=jnp.float32)
            + bl_ref[...])


def _run_final(accp, hs, dinv, cpT, b1, W2, b2, Wl, bl):
    grid = (NPAD // BLK2,)
    return pl.pallas_call(
        _final_body,
        grid=grid,
        in_specs=[
            pl.BlockSpec((NC, BLK2, H), lambda i: (0, i, 0)),
            pl.BlockSpec((BLK2, H), lambda i: (i, 0)),
            pl.BlockSpec((BLK2, 1), lambda i: (i, 0)),
            pl.BlockSpec((BLK2, NC), lambda i: (i, 0)),
            pl.BlockSpec((1, H), lambda i: (0, 0)),
            pl.BlockSpec((H, H), lambda i: (0, 0)),
            pl.BlockSpec((1, H), lambda i: (0, 0)),
            pl.BlockSpec((H, C), lambda i: (0, 0)),
            pl.BlockSpec((1, C), lambda i: (0, 0)),
        ],
        out_specs=pl.BlockSpec((1, C), lambda i: (0, 0)),
        out_shape=jax.ShapeDtypeStruct((1, C), jnp.float32),
        scratch_shapes=[pltpu.VMEM((1, H), jnp.float32)],
    )(accp, hs, dinv, cpT, b1, W2, b2, Wl, bl)


@functools.partial(jax.jit)
def kernel(x, edge_index, W1, b1, W2, b2, Wl, bl):
    ei = edge_index.astype(jnp.int32)
    pad = jnp.full((EPAD - E,), NPAD - 1, jnp.int32)
    srcp = jnp.concatenate([ei[0], pad])
    dstp = jnp.concatenate([ei[1], pad])
    src2 = srcp.reshape(NW, EPT)
    dst2 = dstp.reshape(NW, EPT)
    dst3 = dstp.reshape(NW, DCHUNKS, DCHUNK)
    xp = jnp.pad(x.astype(jnp.float32), ((0, NPAD - N), (0, 0)))

    ones128 = jnp.ones((DCHUNK,), jnp.float32)
    z640 = jnp.zeros((ROWS_PER_TILE,), jnp.float32)
    z2d = jnp.zeros((CHUNK, H), jnp.float32)

    degp = _run_deg(dst3, ones128, z640).reshape(NC, NPAD)
    hs, dinv2 = _run_dense1(xp, W1.astype(jnp.float32), degp.T)
    dinv1 = dinv2.reshape(NPAD)
    accp, cp = _run_msg(hs, dinv1, src2, dst2, z2d)
    cp = cp.reshape(NC, NPAD)
    out = _run_final(accp, hs, dinv2, cp.T,
                     b1.reshape(1, H).astype(jnp.float32),
                     W2.astype(jnp.float32),
                     b2.reshape(1, H).astype(jnp.float32),
                     Wl.astype(jnp.float32),
                     bl.reshape(1, C).astype(jnp.float32))
    return out


# speedup vs baseline: 1.2182x; 1.2182x over previous
import functools

import jax
import jax.numpy as jnp
from jax import lax
from jax.experimental import pallas as pl
from jax.experimental.pallas import tpu as pltpu
from jax.experimental.pallas import tpu_sc as plsc

N = 10000
D = 128
H = 128
C = 2
E = 320000

NC = 2
NS = 16
NW = NC * NS

NPAD = 10240
ROWS_PER_TILE = NPAD // NS
CHUNK = 128
CHUNKS = 79
EPT = CHUNKS * CHUNK
EPAD = EPT * NW
DCHUNKS = 80
EPAD2 = DCHUNKS * CHUNK * NW
WCH = ROWS_PER_TILE // CHUNK


def _sc_mesh():
    return plsc.VectorSubcoreMesh(core_axis_name="c", subcore_axis_name="s",
                                  num_cores=NC, num_subcores=NS)


def _deg_body(dst_hbm, ones_hbm, z640_hbm, degp_hbm,
              deg_sh, idx_v, ones_v, z_v, sem, ssem):
    c = lax.axis_index("c")
    s = lax.axis_index("s")
    w = c * NS + s
    base = s * ROWS_PER_TILE
    pltpu.sync_copy(z640_hbm, z_v)
    pltpu.sync_copy(z_v, deg_sh.at[pl.ds(base, ROWS_PER_TILE)])
    pltpu.sync_copy(ones_hbm, ones_v)
    pltpu.sync_copy(dst_hbm.at[w], idx_v)
    plsc.subcore_barrier()

    def group(g, _):
        def fire(b, carry):
            pltpu.async_copy(ones_v, deg_sh.at[idx_v.at[g * 8 + b]], ssem,
                             add=True)
            return carry

        lax.fori_loop(0, 8, fire, 0)

        def drain(b, carry):
            pltpu.make_async_copy(ones_v, deg_sh.at[idx_v.at[g * 8 + b]],
                                  ssem).wait()
            return carry

        lax.fori_loop(0, 8, drain, 0)
        return _

    lax.fori_loop(0, DCHUNKS // 8, group, 0)
    plsc.subcore_barrier()
    pltpu.sync_copy(deg_sh.at[pl.ds(base, ROWS_PER_TILE)], z_v)
    pltpu.sync_copy(z_v, degp_hbm.at[pl.ds(c * NPAD + base, ROWS_PER_TILE)])


def _run_deg(dst3, ones128, z640):
    k = pl.kernel(
        _deg_body,
        out_type=jax.ShapeDtypeStruct((NC * NPAD,), jnp.float32),
        mesh=_sc_mesh(),
        scratch_types=[
            pltpu.VMEM_SHARED((NPAD,), jnp.float32),
            pltpu.VMEM((DCHUNKS, CHUNK), jnp.int32),
            pltpu.VMEM((CHUNK,), jnp.float32),
            pltpu.VMEM((ROWS_PER_TILE,), jnp.float32),
            pltpu.SemaphoreType.DMA,
            pltpu.SemaphoreType.DMA,
        ],
    )
    return k(dst3, ones128, z640)


def _msg_body(hs_hbm, dinv_hbm, src3_hbm, dst3_hbm, z2d_hbm,
              accp_hbm, cp_hbm,
              acc_sh, cacc_sh,
              src_idx, dst_idx, rows_v, dvals, sem, sem2):
    c = lax.axis_index("c")
    s = lax.axis_index("s")
    w = c * NS + s
    base = s * ROWS_PER_TILE

    pltpu.sync_copy(src3_hbm.at[w], src_idx)
    pltpu.sync_copy(dst3_hbm.at[w], dst_idx)
    pltpu.sync_copy(z2d_hbm, rows_v)
    for k in range(8):
        dvals[pl.ds(k * 16, 16)] = jnp.zeros((16,), jnp.float32)
    for k in range(WCH):
        pltpu.sync_copy(rows_v, acc_sh.at[pl.ds(base + k * CHUNK, CHUNK)])
        pltpu.sync_copy(dvals, cacc_sh.at[pl.ds(base + k * CHUNK, CHUNK)])
    plsc.subcore_barrier()

    def step(j, _):
        gat = pltpu.async_copy(hs_hbm.at[src_idx.at[j]], rows_v, sem)
        pltpu.async_copy(dinv_hbm.at[dst_idx.at[j]], dvals, sem2).wait()
        pltpu.sync_copy(dvals, cacc_sh.at[src_idx.at[j]], add=True)
        gat.wait()
        pltpu.sync_copy(rows_v, acc_sh.at[dst_idx.at[j]], add=True)
        return _

    lax.fori_loop(0, CHUNKS, step, 0)
    plsc.subcore_barrier()

    for k in range(WCH):
        r = base + k * CHUNK
        pltpu.sync_copy(acc_sh.at[pl.ds(r, CHUNK)], rows_v)
        pltpu.sync_copy(rows_v, accp_hbm.at[c, pl.ds(r, CHUNK)])
        pltpu.sync_copy(cacc_sh.at[pl.ds(r, CHUNK)], dvals)
        pltpu.sync_copy(dvals, cp_hbm.at[pl.ds(c * NPAD + r, CHUNK)])


def _run_msg(hs, dinv, src3, dst3, z2d):
    k = pl.kernel(
        _msg_body,
        out_type=(
            jax.ShapeDtypeStruct((NC, NPAD, H), jnp.float32),
            jax.ShapeDtypeStruct((NC * NPAD,), jnp.float32),
        ),
        mesh=_sc_mesh(),
        scratch_types=[
            pltpu.VMEM_SHARED((NPAD, H), jnp.float32),
            pltpu.VMEM_SHARED((NPAD,), jnp.float32),
            pltpu.VMEM((CHUNKS, CHUNK), jnp.int32),
            pltpu.VMEM((CHUNKS, CHUNK), jnp.int32),
            pltpu.VMEM((CHUNK, H), jnp.float32),
            pltpu.VMEM((CHUNK,), jnp.float32),
            pltpu.SemaphoreType.DMA,
            pltpu.SemaphoreType.DMA,
        ],
    )
    return k(hs, dinv, src3, dst3, z2d)


BLK1 = 512


def _dense1_body(x_ref, w1_ref, degt_ref, hs_ref, dinv_ref):
    pid = pl.program_id(0)
    deg = degt_ref[:, 0:1] + degt_ref[:, 1:2] + 1.0
    row = pid * BLK1 + lax.broadcasted_iota(jnp.int32, (BLK1, 1), 0)
    dinv = jnp.where(row < N, lax.rsqrt(jnp.maximum(deg, 1.0)), 0.0)
    dinv_ref[...] = dinv
    h = jnp.dot(x_ref[...], w1_ref[...],
                preferred_element_type=jnp.float32,
                precision=jax.lax.Precision.HIGHEST)
    hs_ref[...] = h * dinv


def _run_dense1(xp, W1, degT):
    grid = (NPAD // BLK1,)
    return pl.pallas_call(
        _dense1_body,
        grid=grid,
        in_specs=[
            pl.BlockSpec((BLK1, D), lambda i: (i, 0)),
            pl.BlockSpec((D, H), lambda i: (0, 0)),
            pl.BlockSpec((BLK1, NC), lambda i: (i, 0)),
        ],
        out_specs=[
            pl.BlockSpec((BLK1, H), lambda i: (i, 0)),
            pl.BlockSpec((BLK1, 1), lambda i: (i, 0)),
        ],
        out_shape=[
            jax.ShapeDtypeStruct((NPAD, H), jnp.float32),
            jax.ShapeDtypeStruct((NPAD, 1), jnp.float32),
        ],
    )(xp, W1, degT)


BLK2 = 512


def _final_body(accp_ref, hs_ref, dinv_ref, cpt_ref, b1_ref, w2_ref, b2_ref,
                wl_ref, bl_ref, out_ref, g_ref):
    i = pl.program_id(0)

    @pl.when(i == 0)
    def _init():
        g_ref[...] = jnp.zeros_like(g_ref)

    acc = accp_ref[0] + accp_ref[1] + hs_ref[...]
    dinv = dinv_ref[...]
    h1 = jnp.maximum(dinv * acc + b1_ref[...], 0.0)
    cvec = dinv * (cpt_ref[:, 0:1] + cpt_ref[:, 1:2] + dinv)
    g_ref[...] += jnp.sum(h1 * cvec, axis=0, keepdims=True)

    @pl.when(i == pl.num_programs(0) - 1)
    def _fin():
        g = g_ref[...] * (1.0 / N)
        t = jnp.dot(g, w2_ref[...], preferred_element_type=jnp.float32)
        t = t + b2_ref[...]
        out_ref[...] = (
            jnp.dot(t, wl_ref[...], preferred_element_type=jnp.float32)
            + bl_ref[...])


def _run_final(accp, hs, dinv, cpT, b1, W2, b2, Wl, bl):
    grid = (NPAD // BLK2,)
    return pl.pallas_call(
        _final_body,
        grid=grid,
        in_specs=[
            pl.BlockSpec((NC, BLK2, H), lambda i: (0, i, 0)),
            pl.BlockSpec((BLK2, H), lambda i: (i, 0)),
            pl.BlockSpec((BLK2, 1), lambda i: (i, 0)),
            pl.BlockSpec((BLK2, NC), lambda i: (i, 0)),
            pl.BlockSpec((1, H), lambda i: (0, 0)),
            pl.BlockSpec((H, H), lambda i: (0, 0)),
            pl.BlockSpec((1, H), lambda i: (0, 0)),
            pl.BlockSpec((H, C), lambda i: (0, 0)),
            pl.BlockSpec((1, C), lambda i: (0, 0)),
        ],
        out_specs=pl.BlockSpec((1, C), lambda i: (0, 0)),
        out_shape=jax.ShapeDtypeStruct((1, C), jnp.float32),
        scratch_shapes=[pltpu.VMEM((1, H), jnp.float32)],
    )(accp, hs, dinv, cpT, b1, W2, b2, Wl, bl)


@functools.partial(jax.jit)
def kernel(x, edge_index, W1, b1, W2, b2, Wl, bl):
    ei = edge_index.astype(jnp.int32)
    pad = jnp.full((EPAD - E,), NPAD - 1, jnp.int32)
    pad2 = jnp.full((EPAD2 - E,), NPAD - 1, jnp.int32)
    src3 = jnp.concatenate([ei[0], pad]).reshape(NW, CHUNKS, CHUNK)
    dst3 = jnp.concatenate([ei[1], pad]).reshape(NW, CHUNKS, CHUNK)
    dst3d = jnp.concatenate([ei[1], pad2]).reshape(NW, DCHUNKS, CHUNK)
    xp = jnp.pad(x.astype(jnp.float32), ((0, NPAD - N), (0, 0)))

    ones128 = jnp.ones((CHUNK,), jnp.float32)
    z640 = jnp.zeros((ROWS_PER_TILE,), jnp.float32)
    z2d = jnp.zeros((CHUNK, H), jnp.float32)

    degp = _run_deg(dst3d, ones128, z640).reshape(NC, NPAD)
    hs, dinv2 = _run_dense1(xp, W1.astype(jnp.float32), degp.T)
    dinv1 = dinv2.reshape(NPAD)
    accp, cp = _run_msg(hs, dinv1, src3, dst3, z2d)
    cp = cp.reshape(NC, NPAD)
    out = _run_final(accp, hs, dinv2, cp.T,
                     b1.reshape(1, H).astype(jnp.float32),
                     W2.astype(jnp.float32),
                     b2.reshape(1, H).astype(jnp.float32),
                     Wl.astype(jnp.float32),
                     bl.reshape(1, C).astype(jnp.float32))
    return out
